# Initial kernel scaffold; baseline (speedup 1.0000x reference)
import jax
import jax.numpy as jnp
from jax.experimental import pallas as pl


def kernel(x, edge_index, batch, W1, b1, W2, b2, W3, b3, Wf1, bf1, Wf2, bf2):
    raise NotImplementedError("write your pallas kernel here")


# speedup vs baseline: 3.6167x; 3.6167x over previous
import functools

import jax
import jax.numpy as jnp
from jax import lax
from jax.experimental import pallas as pl
from jax.experimental.pallas import tpu as pltpu
from jax.experimental.pallas import tpu_sc as plsc

_N = 10000
_E = 160000
_G = 16
_NC = 2
_NS = 16
_NW = _NC * _NS
_BATCH = 128
_NB = 40
_EPW = _BATCH * _NB
_EPAD = _EPW * _NW
_RP = 10112
_STR = _RP // _NS

_mesh = plsc.VectorSubcoreMesh(core_axis_name="c", subcore_axis_name="s")


def _zero_stripe(zbuf, acc, w0):
    nfull = _STR // _BATCH
    for k in range(nfull):
        pltpu.sync_copy(zbuf, acc.at[pl.ds(w0 + k * _BATCH, _BATCH)])
    rem = _STR - nfull * _BATCH
    if rem:
        pltpu.sync_copy(zbuf.at[pl.ds(0, rem)],
                        acc.at[pl.ds(w0 + nfull * _BATCH, rem)])


def _seg_sum_sc(pp, srcp, dstp, nchunks):
    zeros = jnp.zeros((_BATCH, 128), jnp.float32)

    def body(pp_ref, src_ref, dst_ref, z_ref, out_ref,
             src_v, dst_v, rows_v, zbuf, acc):
        cid = lax.axis_index("c")
        sid = lax.axis_index("s")
        wid = cid * _NS + sid
        pltpu.sync_copy(src_ref.at[wid], src_v)
        pltpu.sync_copy(dst_ref.at[wid], dst_v)
        pltpu.sync_copy(z_ref, zbuf)
        w0 = sid * _STR
        _zero_stripe(zbuf, acc, w0)
        plsc.subcore_barrier()
        for c in range(nchunks):
            def jbody(j, carry):
                pltpu.sync_copy(pp_ref.at[c].at[src_v.at[j]], rows_v)
                pltpu.sync_copy(rows_v, acc.at[dst_v.at[j]], add=True)
                return carry
            lax.fori_loop(0, _NB, jbody, 0)
            plsc.subcore_barrier()
            pltpu.sync_copy(acc.at[pl.ds(w0, _STR)],
                            out_ref.at[cid, c, pl.ds(w0, _STR)])
            _zero_stripe(zbuf, acc, w0)
            plsc.subcore_barrier()

    kfn = pl.kernel(
        body,
        out_type=jax.ShapeDtypeStruct((_NC, nchunks, _RP, 128), jnp.float32),
        mesh=_mesh,
        scratch_types=[
            pltpu.VMEM((_NB, _BATCH), jnp.int32),
            pltpu.VMEM((_NB, _BATCH), jnp.int32),
            pltpu.VMEM((_BATCH, 128), jnp.float32),
            pltpu.VMEM((_BATCH, 128), jnp.float32),
            pltpu.VMEM_SHARED((_RP, 128), jnp.float32),
        ],
    )
    return kfn(pp, srcp, dstp, zeros)


def _deg_sc(dstp):
    ones = jnp.ones((_BATCH, 128), jnp.float32)
    zeros = jnp.zeros((_BATCH, 128), jnp.float32)

    def body(dst_ref, ones_ref, z_ref, out_ref, dst_v, ones_v, zbuf, acc):
        cid = lax.axis_index("c")
        sid = lax.axis_index("s")
        wid = cid * _NS + sid
        pltpu.sync_copy(dst_ref.at[wid], dst_v)
        pltpu.sync_copy(ones_ref, ones_v)
        pltpu.sync_copy(z_ref, zbuf)
        w0 = sid * _STR
        _zero_stripe(zbuf, acc, w0)
        plsc.subcore_barrier()

        def jbody(j, carry):
            pltpu.sync_copy(ones_v, acc.at[dst_v.at[j]], add=True)
            return carry
        lax.fori_loop(0, _NB, jbody, 0)
        plsc.subcore_barrier()
        pltpu.sync_copy(acc.at[pl.ds(w0, _STR)],
                        out_ref.at[cid, pl.ds(w0, _STR)])

    kfn = pl.kernel(
        body,
        out_type=jax.ShapeDtypeStruct((_NC, _RP, 128), jnp.float32),
        mesh=_mesh,
        scratch_types=[
            pltpu.VMEM((_NB, _BATCH), jnp.int32),
            pltpu.VMEM((_BATCH, 128), jnp.float32),
            pltpu.VMEM((_BATCH, 128), jnp.float32),
            pltpu.VMEM_SHARED((_RP, 128), jnp.float32),
        ],
    )
    return kfn(dstp, ones, zeros)


_BM = 1000
_GRID = _N // _BM


def _dinv_of(degp_ref):
    deg = 1.0 + degp_ref[0, :, 0] + degp_ref[1, :, 0]
    return lax.rsqrt(deg)


def _k1_body(x_ref, degp_ref, w_ref, out_ref):
    dinv = _dinv_of(degp_ref)
    p = jnp.dot(x_ref[...], w_ref[...], preferred_element_type=jnp.float32)
    pp = p * dinv[:, None]
    for c in range(4):
        out_ref[c] = pp[:, c * 128:(c + 1) * 128]


def _k1(x, degp, W1):
    return pl.pallas_call(
        _k1_body,
        grid=(_GRID,),
        in_specs=[
            pl.BlockSpec((_BM, 256), lambda i: (i, 0)),
            pl.BlockSpec((_NC, _BM, 128), lambda i: (0, i, 0)),
            pl.BlockSpec((256, 512), lambda i: (0, 0)),
        ],
        out_specs=pl.BlockSpec((4, _BM, 128), lambda i: (0, i, 0)),
        out_shape=jax.ShapeDtypeStruct((4, _N, 128), jnp.float32),
    )(x, degp, W1)


def _mid_body(nc_in, nc_out, b_ref, pp_ref, degp_ref, w_ref, bias_ref, out_ref):
    dinv = _dinv_of(degp_ref)
    acc = None
    for c in range(nc_in):
        hc = b_ref[0, c] + b_ref[1, c] + pp_ref[c]
        hc = jnp.maximum(hc * dinv[:, None] + bias_ref[0, c * 128:(c + 1) * 128], 0.0)
        term = jnp.dot(hc, w_ref[c * 128:(c + 1) * 128, :],
                       preferred_element_type=jnp.float32)
        acc = term if acc is None else acc + term
    ppo = acc * dinv[:, None]
    for c in range(nc_out):
        out_ref[c] = ppo[:, c * 128:(c + 1) * 128]


def _kmid(B, pp, degp, W, bias, nc_in, nc_out):
    return pl.pallas_call(
        functools.partial(_mid_body, nc_in, nc_out),
        grid=(_GRID,),
        in_specs=[
            pl.BlockSpec((_NC, nc_in, _BM, 128), lambda i: (0, 0, i, 0)),
            pl.BlockSpec((nc_in, _BM, 128), lambda i: (0, i, 0)),
            pl.BlockSpec((_NC, _BM, 128), lambda i: (0, i, 0)),
            pl.BlockSpec((nc_in * 128, nc_out * 128), lambda i: (0, 0)),
            pl.BlockSpec((1, nc_in * 128), lambda i: (0, 0)),
        ],
        out_specs=pl.BlockSpec((nc_out, _BM, 128), lambda i: (0, i, 0)),
        out_shape=jax.ShapeDtypeStruct((nc_out, _N, 128), jnp.float32),
    )(B, pp, degp, W, bias)


def _k4_body(b_ref, pp_ref, degp_ref, bias_ref, batch_ref,
             wf1_ref, bf1_ref, wf2_ref, bf2_ref, out_ref, accp, accc):
    i = pl.program_id(0)

    @pl.when(i == 0)
    def _():
        accp[...] = jnp.zeros((_G, 256), jnp.float32)
        accc[...] = jnp.zeros((_G, 128), jnp.float32)

    dinv = _dinv_of(degp_ref)
    gids = lax.broadcasted_iota(jnp.int32, (_G, _BM), 0)
    oh = (batch_ref[0, 0][None, :] == gids).astype(jnp.float32)
    for c in range(2):
        hc = b_ref[0, c] + b_ref[1, c] + pp_ref[c]
        hc = hc * dinv[:, None] + bias_ref[0, c * 128:(c + 1) * 128]
        accp[:, c * 128:(c + 1) * 128] += jnp.dot(
            oh, hc, preferred_element_type=jnp.float32)
    cnt = jnp.sum(oh, axis=1, keepdims=True)
    accc[...] += jnp.broadcast_to(cnt, (_G, 128))

    @pl.when(i == _GRID - 1)
    def _():
        pooled = accp[...] / jnp.maximum(accc[:, 0:1], 1.0)
        f = jnp.dot(pooled, wf1_ref[...], preferred_element_type=jnp.float32)
        f = jnp.maximum(f + bf1_ref[...], 0.0)
        f2 = jnp.dot(f, wf2_ref[...], preferred_element_type=jnp.float32)
        f2 = f2 + bf2_ref[...]
        m = jnp.max(f2, axis=1, keepdims=True)
        lse = jnp.log(jnp.sum(jnp.exp(f2 - m), axis=1, keepdims=True)) + m
        out_ref[...] = f2 - lse


def _k4(B3, pp3, degp, bias3, batch_r, Wf1, bf1, Wf2, bf2):
    return pl.pallas_call(
        _k4_body,
        grid=(_GRID,),
        in_specs=[
            pl.BlockSpec((_NC, 2, _BM, 128), lambda i: (0, 0, i, 0)),
            pl.BlockSpec((2, _BM, 128), lambda i: (0, i, 0)),
            pl.BlockSpec((_NC, _BM, 128), lambda i: (0, i, 0)),
            pl.BlockSpec((1, 256), lambda i: (0, 0)),
            pl.BlockSpec((1, 1, _BM), lambda i: (i, 0, 0)),
            pl.BlockSpec((256, 512), lambda i: (0, 0)),
            pl.BlockSpec((1, 512), lambda i: (0, 0)),
            pl.BlockSpec((512, 128), lambda i: (0, 0)),
            pl.BlockSpec((1, 128), lambda i: (0, 0)),
        ],
        out_specs=pl.BlockSpec((_G, 128), lambda i: (0, 0)),
        out_shape=jax.ShapeDtypeStruct((_G, 128), jnp.float32),
        scratch_shapes=[
            pltpu.VMEM((_G, 256), jnp.float32),
            pltpu.VMEM((_G, 128), jnp.float32),
        ],
    )(B3, pp3, degp, bias3, batch_r, Wf1, bf1, Wf2, bf2)


def kernel(x, edge_index, batch, W1, b1, W2, b2, W3, b3, Wf1, bf1, Wf2, bf2):
    src, dst = edge_index[0], edge_index[1]
    npad = _EPAD - _E
    srcp = jnp.concatenate([src, jnp.zeros((npad,), jnp.int32)]
                           ).reshape(_NW, _NB, _BATCH)
    dstp = jnp.concatenate([dst, jnp.full((npad,), _N, jnp.int32)]
                           ).reshape(_NW, _NB, _BATCH)

    degp = _deg_sc(dstp)

    pp1 = _k1(x, degp, W1)
    B1 = _seg_sum_sc(pp1, srcp, dstp, 4)
    pp2 = _kmid(B1, pp1, degp, W2, b1.reshape(1, 512), 4, 4)
    B2 = _seg_sum_sc(pp2, srcp, dstp, 4)
    pp3 = _kmid(B2, pp2, degp, W3, b2.reshape(1, 512), 4, 2)
    B3 = _seg_sum_sc(pp3, srcp, dstp, 2)

    batch_r = batch.reshape(_GRID, 1, _BM)
    return _k4(B3, pp3, degp, b3.reshape(1, 256), batch_r,
               Wf1, bf1.reshape(1, 512), Wf2, bf2.reshape(1, 128))


# speedup vs baseline: 3.8865x; 1.0746x over previous
import functools

import jax
import jax.numpy as jnp
from jax import lax
from jax.experimental import pallas as pl
from jax.experimental.pallas import tpu as pltpu
from jax.experimental.pallas import tpu_sc as plsc

_N = 10000
_E = 160000
_G = 16
_NC = 2
_NS = 16
_NW = _NC * _NS
_BATCH = 128
_NB = 40
_EPW = _BATCH * _NB
_EPAD = _EPW * _NW
_RP = 10112
_STR = _RP // _NS

_NBUF = 2

_mesh = plsc.VectorSubcoreMesh(core_axis_name="c", subcore_axis_name="s")


_ZROWS = 32


def _zero_stripe(zbuf, acc, w0):
    nfull = _STR // _ZROWS
    for k in range(nfull):
        pltpu.sync_copy(zbuf, acc.at[pl.ds(w0 + k * _ZROWS, _ZROWS)])
    rem = _STR - nfull * _ZROWS
    if rem:
        pltpu.sync_copy(zbuf.at[pl.ds(0, rem)],
                        acc.at[pl.ds(w0 + nfull * _ZROWS, rem)])


def _seg_sum_sc(pp, srcp, dstp, nchunks):
    zeros = jnp.zeros((_ZROWS, 128), jnp.float32)

    def body(pp_ref, src_ref, dst_ref, z_ref, out_ref,
             src_v, dst_v, rows_v, zbuf, acc, gsem, ssem):
        cid = lax.axis_index("c")
        sid = lax.axis_index("s")
        wid = cid * _NS + sid
        pltpu.sync_copy(src_ref.at[wid], src_v)
        pltpu.sync_copy(dst_ref.at[wid], dst_v)
        pltpu.sync_copy(z_ref, zbuf)
        w0 = sid * _STR
        _zero_stripe(zbuf, acc, w0)
        plsc.subcore_barrier()

        def gather_start(c, j, b):
            pltpu.async_copy(pp_ref.at[c].at[src_v.at[j]],
                             rows_v.at[b], gsem.at[b])

        def gather_wait(c, b):
            pltpu.make_async_copy(pp_ref.at[c].at[src_v.at[0]],
                                  rows_v.at[b], gsem.at[b]).wait()

        def scat_start(j, b):
            pltpu.async_copy(rows_v.at[b], acc.at[dst_v.at[j]],
                             ssem.at[b], add=True)

        def scat_wait(b):
            pltpu.make_async_copy(rows_v.at[b], acc.at[dst_v.at[0]],
                                  ssem.at[b]).wait()

        for c in range(nchunks):
            for p in range(_NBUF - 1):
                gather_start(c, p, p)

            def jbody(j, carry):
                b = lax.rem(j, _NBUF)
                nxt = j + _NBUF - 1
                bn = lax.rem(nxt, _NBUF)

                @pl.when(nxt < _NB)
                def _():
                    @pl.when(j >= 1)
                    def _():
                        scat_wait(bn)
                    gather_start(c, nxt, bn)

                gather_wait(c, b)
                scat_start(j, b)
                return carry
            lax.fori_loop(0, _NB, jbody, 0)
            for p in range(_NBUF):
                scat_wait((_NB - _NBUF + p) % _NBUF)
            plsc.subcore_barrier()
            pltpu.sync_copy(acc.at[pl.ds(w0, _STR)],
                            out_ref.at[cid, c, pl.ds(w0, _STR)])
            _zero_stripe(zbuf, acc, w0)
            plsc.subcore_barrier()

    kfn = pl.kernel(
        body,
        out_type=jax.ShapeDtypeStruct((_NC, nchunks, _RP, 128), jnp.float32),
        mesh=_mesh,
        scratch_types=[
            pltpu.VMEM((_NB, _BATCH), jnp.int32),
            pltpu.VMEM((_NB, _BATCH), jnp.int32),
            pltpu.VMEM((_NBUF, _BATCH, 128), jnp.float32),
            pltpu.VMEM((_ZROWS, 128), jnp.float32),
            pltpu.VMEM_SHARED((_RP, 128), jnp.float32),
            pltpu.SemaphoreType.DMA((_NBUF,)),
            pltpu.SemaphoreType.DMA((_NBUF,)),
        ],
    )
    return kfn(pp, srcp, dstp, zeros)


def _deg_sc(dstp):
    ones = jnp.ones((_BATCH, 128), jnp.float32)
    zeros = jnp.zeros((_ZROWS, 128), jnp.float32)

    def body(dst_ref, ones_ref, z_ref, out_ref, dst_v, ones_v, zbuf, acc):
        cid = lax.axis_index("c")
        sid = lax.axis_index("s")
        wid = cid * _NS + sid
        pltpu.sync_copy(dst_ref.at[wid], dst_v)
        pltpu.sync_copy(ones_ref, ones_v)
        pltpu.sync_copy(z_ref, zbuf)
        w0 = sid * _STR
        _zero_stripe(zbuf, acc, w0)
        plsc.subcore_barrier()

        def jbody(j, carry):
            pltpu.sync_copy(ones_v, acc.at[dst_v.at[j]], add=True)
            return carry
        lax.fori_loop(0, _NB, jbody, 0)
        plsc.subcore_barrier()
        pltpu.sync_copy(acc.at[pl.ds(w0, _STR)],
                        out_ref.at[cid, pl.ds(w0, _STR)])

    kfn = pl.kernel(
        body,
        out_type=jax.ShapeDtypeStruct((_NC, _RP, 128), jnp.float32),
        mesh=_mesh,
        scratch_types=[
            pltpu.VMEM((_NB, _BATCH), jnp.int32),
            pltpu.VMEM((_BATCH, 128), jnp.float32),
            pltpu.VMEM((_ZROWS, 128), jnp.float32),
            pltpu.VMEM_SHARED((_RP, 128), jnp.float32),
        ],
    )
    return kfn(dstp, ones, zeros)


_BM = 1000
_GRID = _N // _BM


def _dinv_of(degp_ref):
    deg = 1.0 + degp_ref[0, :, 0] + degp_ref[1, :, 0]
    return lax.rsqrt(deg)


def _k1_body(x_ref, degp_ref, w_ref, out_ref):
    dinv = _dinv_of(degp_ref)
    p = jnp.dot(x_ref[...], w_ref[...], preferred_element_type=jnp.float32)
    pp = p * dinv[:, None]
    for c in range(4):
        out_ref[c] = pp[:, c * 128:(c + 1) * 128]


def _k1(x, degp, W1):
    return pl.pallas_call(
        _k1_body,
        grid=(_GRID,),
        in_specs=[
            pl.BlockSpec((_BM, 256), lambda i: (i, 0)),
            pl.BlockSpec((_NC, _BM, 128), lambda i: (0, i, 0)),
            pl.BlockSpec((256, 512), lambda i: (0, 0)),
        ],
        out_specs=pl.BlockSpec((4, _BM, 128), lambda i: (0, i, 0)),
        out_shape=jax.ShapeDtypeStruct((4, _N, 128), jnp.float32),
    )(x, degp, W1)


def _mid_body(nc_in, nc_out, b_ref, pp_ref, degp_ref, w_ref, bias_ref, out_ref):
    dinv = _dinv_of(degp_ref)
    acc = None
    for c in range(nc_in):
        hc = b_ref[0, c] + b_ref[1, c] + pp_ref[c]
        hc = jnp.maximum(hc * dinv[:, None] + bias_ref[0, c * 128:(c + 1) * 128], 0.0)
        term = jnp.dot(hc, w_ref[c * 128:(c + 1) * 128, :],
                       preferred_element_type=jnp.float32)
        acc = term if acc is None else acc + term
    ppo = acc * dinv[:, None]
    for c in range(nc_out):
        out_ref[c] = ppo[:, c * 128:(c + 1) * 128]


def _kmid(B, pp, degp, W, bias, nc_in, nc_out):
    return pl.pallas_call(
        functools.partial(_mid_body, nc_in, nc_out),
        grid=(_GRID,),
        in_specs=[
            pl.BlockSpec((_NC, nc_in, _BM, 128), lambda i: (0, 0, i, 0)),
            pl.BlockSpec((nc_in, _BM, 128), lambda i: (0, i, 0)),
            pl.BlockSpec((_NC, _BM, 128), lambda i: (0, i, 0)),
            pl.BlockSpec((nc_in * 128, nc_out * 128), lambda i: (0, 0)),
            pl.BlockSpec((1, nc_in * 128), lambda i: (0, 0)),
        ],
        out_specs=pl.BlockSpec((nc_out, _BM, 128), lambda i: (0, i, 0)),
        out_shape=jax.ShapeDtypeStruct((nc_out, _N, 128), jnp.float32),
    )(B, pp, degp, W, bias)


def _k4_body(b_ref, pp_ref, degp_ref, bias_ref, batch_ref,
             wf1_ref, bf1_ref, wf2_ref, bf2_ref, out_ref, accp, accc):
    i = pl.program_id(0)

    @pl.when(i == 0)
    def _():
        accp[...] = jnp.zeros((_G, 256), jnp.float32)
        accc[...] = jnp.zeros((_G, 128), jnp.float32)

    dinv = _dinv_of(degp_ref)
    gids = lax.broadcasted_iota(jnp.int32, (_G, _BM), 0)
    oh = (batch_ref[0, 0][None, :] == gids).astype(jnp.float32)
    for c in range(2):
        hc = b_ref[0, c] + b_ref[1, c] + pp_ref[c]
        hc = hc * dinv[:, None] + bias_ref[0, c * 128:(c + 1) * 128]
        accp[:, c * 128:(c + 1) * 128] += jnp.dot(
            oh, hc, preferred_element_type=jnp.float32)
    cnt = jnp.sum(oh, axis=1, keepdims=True)
    accc[...] += jnp.broadcast_to(cnt, (_G, 128))

    @pl.when(i == _GRID - 1)
    def _():
        pooled = accp[...] / jnp.maximum(accc[:, 0:1], 1.0)
        f = jnp.dot(pooled, wf1_ref[...], preferred_element_type=jnp.float32)
        f = jnp.maximum(f + bf1_ref[...], 0.0)
        f2 = jnp.dot(f, wf2_ref[...], preferred_element_type=jnp.float32)
        f2 = f2 + bf2_ref[...]
        m = jnp.max(f2, axis=1, keepdims=True)
        lse = jnp.log(jnp.sum(jnp.exp(f2 - m), axis=1, keepdims=True)) + m
        out_ref[...] = f2 - lse


def _k4(B3, pp3, degp, bias3, batch_r, Wf1, bf1, Wf2, bf2):
    return pl.pallas_call(
        _k4_body,
        grid=(_GRID,),
        in_specs=[
            pl.BlockSpec((_NC, 2, _BM, 128), lambda i: (0, 0, i, 0)),
            pl.BlockSpec((2, _BM, 128), lambda i: (0, i, 0)),
            pl.BlockSpec((_NC, _BM, 128), lambda i: (0, i, 0)),
            pl.BlockSpec((1, 256), lambda i: (0, 0)),
            pl.BlockSpec((1, 1, _BM), lambda i: (i, 0, 0)),
            pl.BlockSpec((256, 512), lambda i: (0, 0)),
            pl.BlockSpec((1, 512), lambda i: (0, 0)),
            pl.BlockSpec((512, 128), lambda i: (0, 0)),
            pl.BlockSpec((1, 128), lambda i: (0, 0)),
        ],
        out_specs=pl.BlockSpec((_G, 128), lambda i: (0, 0)),
        out_shape=jax.ShapeDtypeStruct((_G, 128), jnp.float32),
        scratch_shapes=[
            pltpu.VMEM((_G, 256), jnp.float32),
            pltpu.VMEM((_G, 128), jnp.float32),
        ],
    )(B3, pp3, degp, bias3, batch_r, Wf1, bf1, Wf2, bf2)


def kernel(x, edge_index, batch, W1, b1, W2, b2, W3, b3, Wf1, bf1, Wf2, bf2):
    src, dst = edge_index[0], edge_index[1]
    npad = _EPAD - _E
    srcp = jnp.concatenate([src, jnp.zeros((npad,), jnp.int32)]
                           ).reshape(_NW, _NB, _BATCH)
    dstp = jnp.concatenate([dst, jnp.full((npad,), _N, jnp.int32)]
                           ).reshape(_NW, _NB, _BATCH)

    degp = _deg_sc(dstp)

    pp1 = _k1(x, degp, W1)
    B1 = _seg_sum_sc(pp1, srcp, dstp, 4)
    pp2 = _kmid(B1, pp1, degp, W2, b1.reshape(1, 512), 4, 4)
    B2 = _seg_sum_sc(pp2, srcp, dstp, 4)
    pp3 = _kmid(B2, pp2, degp, W3, b2.reshape(1, 512), 4, 2)
    B3 = _seg_sum_sc(pp3, srcp, dstp, 2)

    batch_r = batch.reshape(_GRID, 1, _BM)
    return _k4(B3, pp3, degp, b3.reshape(1, 256), batch_r,
               Wf1, bf1.reshape(1, 512), Wf2, bf2.reshape(1, 128))


# speedup vs baseline: 4.2533x; 1.0944x over previous
import functools

import jax
import jax.numpy as jnp
from jax import lax
from jax.experimental import pallas as pl
from jax.experimental.pallas import tpu as pltpu
from jax.experimental.pallas import tpu_sc as plsc

_N = 10000
_E = 160000
_G = 16
_NC = 2
_NS = 16
_NW = _NC * _NS
_BATCH = 64
_NB = 80
_EPW = _BATCH * _NB
_EPAD = _EPW * _NW
_RP = 10112
_STR = _RP // _NS

_NBUF = 3

_mesh = plsc.VectorSubcoreMesh(core_axis_name="c", subcore_axis_name="s")


_ZROWS = 32


def _zero_stripe(zbuf, acc, w0):
    nfull = _STR // _ZROWS
    for k in range(nfull):
        pltpu.sync_copy(zbuf, acc.at[pl.ds(w0 + k * _ZROWS, _ZROWS)])
    rem = _STR - nfull * _ZROWS
    if rem:
        pltpu.sync_copy(zbuf.at[pl.ds(0, rem)],
                        acc.at[pl.ds(w0 + nfull * _ZROWS, rem)])


def _seg_sum_sc(pp, srcp, dstp, nchunks):
    zeros = jnp.zeros((_ZROWS, 128), jnp.float32)

    def body(pp_ref, src_ref, dst_ref, z_ref, out_ref,
             src_v, dst_v, rows_v, zbuf, acc, gsem, ssem):
        cid = lax.axis_index("c")
        sid = lax.axis_index("s")
        wid = cid * _NS + sid
        pltpu.sync_copy(src_ref.at[wid], src_v)
        pltpu.sync_copy(dst_ref.at[wid], dst_v)
        pltpu.sync_copy(z_ref, zbuf)
        w0 = sid * _STR
        _zero_stripe(zbuf, acc, w0)
        plsc.subcore_barrier()

        def gather_start(c, j, b):
            pltpu.async_copy(pp_ref.at[c].at[src_v.at[j]],
                             rows_v.at[b], gsem.at[b])

        def gather_wait(c, b):
            pltpu.make_async_copy(pp_ref.at[c].at[src_v.at[0]],
                                  rows_v.at[b], gsem.at[b]).wait()

        def scat_start(j, b):
            pltpu.async_copy(rows_v.at[b], acc.at[dst_v.at[j]],
                             ssem.at[b], add=True)

        def scat_wait(b):
            pltpu.make_async_copy(rows_v.at[b], acc.at[dst_v.at[0]],
                                  ssem.at[b]).wait()

        for c in range(nchunks):
            for p in range(_NBUF - 1):
                gather_start(c, p, p)

            def jbody(j, carry):
                b = lax.rem(j, _NBUF)
                nxt = j + _NBUF - 1
                bn = lax.rem(nxt, _NBUF)

                @pl.when(nxt < _NB)
                def _():
                    @pl.when(j >= 1)
                    def _():
                        scat_wait(bn)
                    gather_start(c, nxt, bn)

                gather_wait(c, b)
                scat_start(j, b)
                return carry
            lax.fori_loop(0, _NB, jbody, 0)
            for p in range(_NBUF):
                scat_wait((_NB - _NBUF + p) % _NBUF)
            plsc.subcore_barrier()
            pltpu.sync_copy(acc.at[pl.ds(w0, _STR)],
                            out_ref.at[cid, c, pl.ds(w0, _STR)])
            _zero_stripe(zbuf, acc, w0)
            plsc.subcore_barrier()

    kfn = pl.kernel(
        body,
        out_type=jax.ShapeDtypeStruct((_NC, nchunks, _RP, 128), jnp.float32),
        mesh=_mesh,
        scratch_types=[
            pltpu.VMEM((_NB, _BATCH), jnp.int32),
            pltpu.VMEM((_NB, _BATCH), jnp.int32),
            pltpu.VMEM((_NBUF, _BATCH, 128), jnp.float32),
            pltpu.VMEM((_ZROWS, 128), jnp.float32),
            pltpu.VMEM_SHARED((_RP, 128), jnp.float32),
            pltpu.SemaphoreType.DMA((_NBUF,)),
            pltpu.SemaphoreType.DMA((_NBUF,)),
        ],
    )
    return kfn(pp, srcp, dstp, zeros)


def _deg_sc(dstp):
    ones = jnp.ones((_BATCH, 128), jnp.float32)
    zeros = jnp.zeros((_ZROWS, 128), jnp.float32)

    def body(dst_ref, ones_ref, z_ref, out_ref, dst_v, ones_v, zbuf, acc):
        cid = lax.axis_index("c")
        sid = lax.axis_index("s")
        wid = cid * _NS + sid
        pltpu.sync_copy(dst_ref.at[wid], dst_v)
        pltpu.sync_copy(ones_ref, ones_v)
        pltpu.sync_copy(z_ref, zbuf)
        w0 = sid * _STR
        _zero_stripe(zbuf, acc, w0)
        plsc.subcore_barrier()

        def jbody(j, carry):
            pltpu.sync_copy(ones_v, acc.at[dst_v.at[j]], add=True)
            return carry
        lax.fori_loop(0, _NB, jbody, 0)
        plsc.subcore_barrier()
        pltpu.sync_copy(acc.at[pl.ds(w0, _STR)],
                        out_ref.at[cid, pl.ds(w0, _STR)])

    kfn = pl.kernel(
        body,
        out_type=jax.ShapeDtypeStruct((_NC, _RP, 128), jnp.float32),
        mesh=_mesh,
        scratch_types=[
            pltpu.VMEM((_NB, _BATCH), jnp.int32),
            pltpu.VMEM((_BATCH, 128), jnp.float32),
            pltpu.VMEM((_ZROWS, 128), jnp.float32),
            pltpu.VMEM_SHARED((_RP, 128), jnp.float32),
        ],
    )
    return kfn(dstp, ones, zeros)


_BM = 1000
_GRID = _N // _BM


def _dinv_of(degp_ref):
    deg = 1.0 + degp_ref[0, :, 0] + degp_ref[1, :, 0]
    return lax.rsqrt(deg)


def _k1_body(x_ref, degp_ref, w_ref, out_ref):
    dinv = _dinv_of(degp_ref)
    p = jnp.dot(x_ref[...], w_ref[...], preferred_element_type=jnp.float32)
    pp = p * dinv[:, None]
    for c in range(4):
        out_ref[c] = pp[:, c * 128:(c + 1) * 128]


def _k1(x, degp, W1):
    return pl.pallas_call(
        _k1_body,
        grid=(_GRID,),
        in_specs=[
            pl.BlockSpec((_BM, 256), lambda i: (i, 0)),
            pl.BlockSpec((_NC, _BM, 128), lambda i: (0, i, 0)),
            pl.BlockSpec((256, 512), lambda i: (0, 0)),
        ],
        out_specs=pl.BlockSpec((4, _BM, 128), lambda i: (0, i, 0)),
        out_shape=jax.ShapeDtypeStruct((4, _N, 128), jnp.float32),
    )(x, degp, W1)


def _mid_body(nc_in, nc_out, b_ref, pp_ref, degp_ref, w_ref, bias_ref, out_ref):
    dinv = _dinv_of(degp_ref)
    acc = None
    for c in range(nc_in):
        hc = b_ref[0, c] + b_ref[1, c] + pp_ref[c]
        hc = jnp.maximum(hc * dinv[:, None] + bias_ref[0, c * 128:(c + 1) * 128], 0.0)
        term = jnp.dot(hc, w_ref[c * 128:(c + 1) * 128, :],
                       preferred_element_type=jnp.float32)
        acc = term if acc is None else acc + term
    ppo = acc * dinv[:, None]
    for c in range(nc_out):
        out_ref[c] = ppo[:, c * 128:(c + 1) * 128]


def _kmid(B, pp, degp, W, bias, nc_in, nc_out):
    return pl.pallas_call(
        functools.partial(_mid_body, nc_in, nc_out),
        grid=(_GRID,),
        in_specs=[
            pl.BlockSpec((_NC, nc_in, _BM, 128), lambda i: (0, 0, i, 0)),
            pl.BlockSpec((nc_in, _BM, 128), lambda i: (0, i, 0)),
            pl.BlockSpec((_NC, _BM, 128), lambda i: (0, i, 0)),
            pl.BlockSpec((nc_in * 128, nc_out * 128), lambda i: (0, 0)),
            pl.BlockSpec((1, nc_in * 128), lambda i: (0, 0)),
        ],
        out_specs=pl.BlockSpec((nc_out, _BM, 128), lambda i: (0, i, 0)),
        out_shape=jax.ShapeDtypeStruct((nc_out, _N, 128), jnp.float32),
    )(B, pp, degp, W, bias)


def _k4_body(b_ref, pp_ref, degp_ref, bias_ref, batch_ref,
             wf1_ref, bf1_ref, wf2_ref, bf2_ref, out_ref, accp, accc):
    i = pl.program_id(0)

    @pl.when(i == 0)
    def _():
        accp[...] = jnp.zeros((_G, 256), jnp.float32)
        accc[...] = jnp.zeros((_G, 128), jnp.float32)

    dinv = _dinv_of(degp_ref)
    gids = lax.broadcasted_iota(jnp.int32, (_G, _BM), 0)
    oh = (batch_ref[0, 0][None, :] == gids).astype(jnp.float32)
    for c in range(2):
        hc = b_ref[0, c] + b_ref[1, c] + pp_ref[c]
        hc = hc * dinv[:, None] + bias_ref[0, c * 128:(c + 1) * 128]
        accp[:, c * 128:(c + 1) * 128] += jnp.dot(
            oh, hc, preferred_element_type=jnp.float32)
    cnt = jnp.sum(oh, axis=1, keepdims=True)
    accc[...] += jnp.broadcast_to(cnt, (_G, 128))

    @pl.when(i == _GRID - 1)
    def _():
        pooled = accp[...] / jnp.maximum(accc[:, 0:1], 1.0)
        f = jnp.dot(pooled, wf1_ref[...], preferred_element_type=jnp.float32)
        f = jnp.maximum(f + bf1_ref[...], 0.0)
        f2 = jnp.dot(f, wf2_ref[...], preferred_element_type=jnp.float32)
        f2 = f2 + bf2_ref[...]
        m = jnp.max(f2, axis=1, keepdims=True)
        lse = jnp.log(jnp.sum(jnp.exp(f2 - m), axis=1, keepdims=True)) + m
        out_ref[...] = f2 - lse


def _k4(B3, pp3, degp, bias3, batch_r, Wf1, bf1, Wf2, bf2):
    return pl.pallas_call(
        _k4_body,
        grid=(_GRID,),
        in_specs=[
            pl.BlockSpec((_NC, 2, _BM, 128), lambda i: (0, 0, i, 0)),
            pl.BlockSpec((2, _BM, 128), lambda i: (0, i, 0)),
            pl.BlockSpec((_NC, _BM, 128), lambda i: (0, i, 0)),
            pl.BlockSpec((1, 256), lambda i: (0, 0)),
            pl.BlockSpec((1, 1, _BM), lambda i: (i, 0, 0)),
            pl.BlockSpec((256, 512), lambda i: (0, 0)),
            pl.BlockSpec((1, 512), lambda i: (0, 0)),
            pl.BlockSpec((512, 128), lambda i: (0, 0)),
            pl.BlockSpec((1, 128), lambda i: (0, 0)),
        ],
        out_specs=pl.BlockSpec((_G, 128), lambda i: (0, 0)),
        out_shape=jax.ShapeDtypeStruct((_G, 128), jnp.float32),
        scratch_shapes=[
            pltpu.VMEM((_G, 256), jnp.float32),
            pltpu.VMEM((_G, 128), jnp.float32),
        ],
    )(B3, pp3, degp, bias3, batch_r, Wf1, bf1, Wf2, bf2)


def kernel(x, edge_index, batch, W1, b1, W2, b2, W3, b3, Wf1, bf1, Wf2, bf2):
    src, dst = edge_index[0], edge_index[1]
    npad = _EPAD - _E
    srcp = jnp.concatenate([src, jnp.zeros((npad,), jnp.int32)]
                           ).reshape(_NW, _NB, _BATCH)
    dstp = jnp.concatenate([dst, jnp.full((npad,), _N, jnp.int32)]
                           ).reshape(_NW, _NB, _BATCH)

    degp = _deg_sc(dstp)

    pp1 = _k1(x, degp, W1)
    B1 = _seg_sum_sc(pp1, srcp, dstp, 4)
    pp2 = _kmid(B1, pp1, degp, W2, b1.reshape(1, 512), 4, 4)
    B2 = _seg_sum_sc(pp2, srcp, dstp, 4)
    pp3 = _kmid(B2, pp2, degp, W3, b2.reshape(1, 512), 4, 2)
    B3 = _seg_sum_sc(pp3, srcp, dstp, 2)

    batch_r = batch.reshape(_GRID, 1, _BM)
    return _k4(B3, pp3, degp, b3.reshape(1, 256), batch_r,
               Wf1, bf1.reshape(1, 512), Wf2, bf2.reshape(1, 128))


# speedup vs baseline: 5.2663x; 1.2381x over previous
import functools

import jax
import jax.numpy as jnp
from jax import lax
from jax.experimental import pallas as pl
from jax.experimental.pallas import tpu as pltpu
from jax.experimental.pallas import tpu_sc as plsc

_N = 10000
_E = 160000
_G = 16
_NC = 2
_NS = 16
_NW = _NC * _NS
_BATCH = 64
_NB = 80
_EPW = _BATCH * _NB
_EPAD = _EPW * _NW
_RP = 10112
_STR = _RP // _NS

_NBUF = 3

_mesh = plsc.VectorSubcoreMesh(core_axis_name="c", subcore_axis_name="s")


_ZROWS = 32


def _zero_stripe(zbuf, acc, w0):
    nfull = _STR // _ZROWS
    for k in range(nfull):
        pltpu.sync_copy(zbuf, acc.at[pl.ds(w0 + k * _ZROWS, _ZROWS)])
    rem = _STR - nfull * _ZROWS
    if rem:
        pltpu.sync_copy(zbuf.at[pl.ds(0, rem)],
                        acc.at[pl.ds(w0 + nfull * _ZROWS, rem)])


def _seg_sum_sc(pp, srcp, dstp, nchunks):
    zeros = jnp.zeros((_ZROWS, 128), jnp.float32)

    def body(pp_ref, src_ref, dst_ref, z_ref, out_ref,
             src_v, dst_v, rows_v, zbuf, acc, gsem, ssem):
        cid = lax.axis_index("c")
        sid = lax.axis_index("s")
        wid = cid * _NS + sid
        pltpu.sync_copy(src_ref.at[wid], src_v)
        pltpu.sync_copy(dst_ref.at[wid], dst_v)
        pltpu.sync_copy(z_ref, zbuf)
        w0 = sid * _STR
        _zero_stripe(zbuf, acc, w0)
        plsc.subcore_barrier()

        def gather_start(c, j, b):
            pltpu.async_copy(pp_ref.at[c].at[src_v.at[j]],
                             rows_v.at[b], gsem.at[b])

        def gather_wait(c, b):
            pltpu.make_async_copy(pp_ref.at[c].at[src_v.at[0]],
                                  rows_v.at[b], gsem.at[b]).wait()

        def scat_start(j, b):
            pltpu.async_copy(rows_v.at[b], acc.at[dst_v.at[j]],
                             ssem.at[b], add=True)

        def scat_wait(b):
            pltpu.make_async_copy(rows_v.at[b], acc.at[dst_v.at[0]],
                                  ssem.at[b]).wait()

        for c in range(nchunks):
            for p in range(_NBUF - 1):
                gather_start(c, p, p)

            def jbody(j, carry):
                b = lax.rem(j, _NBUF)
                nxt = j + _NBUF - 1
                bn = lax.rem(nxt, _NBUF)

                @pl.when(nxt < _NB)
                def _():
                    @pl.when(j >= 1)
                    def _():
                        scat_wait(bn)
                    gather_start(c, nxt, bn)

                gather_wait(c, b)
                scat_start(j, b)
                return carry
            lax.fori_loop(0, _NB, jbody, 0)
            for p in range(_NBUF):
                scat_wait((_NB - _NBUF + p) % _NBUF)
            plsc.subcore_barrier()
            pltpu.sync_copy(acc.at[pl.ds(w0, _STR)],
                            out_ref.at[cid, c, pl.ds(w0, _STR)])
            _zero_stripe(zbuf, acc, w0)
            plsc.subcore_barrier()

    kfn = pl.kernel(
        body,
        out_type=jax.ShapeDtypeStruct((_NC, nchunks, _RP, 128), jnp.float32),
        mesh=_mesh,
        scratch_types=[
            pltpu.VMEM((_NB, _BATCH), jnp.int32),
            pltpu.VMEM((_NB, _BATCH), jnp.int32),
            pltpu.VMEM((_NBUF, _BATCH, 128), jnp.float32),
            pltpu.VMEM((_ZROWS, 128), jnp.float32),
            pltpu.VMEM_SHARED((_RP, 128), jnp.float32),
            pltpu.SemaphoreType.DMA((_NBUF,)),
            pltpu.SemaphoreType.DMA((_NBUF,)),
        ],
    )
    return kfn(pp, srcp, dstp, zeros)


def _deg_sc(dstp):
    ones = jnp.ones((_BATCH, 128), jnp.float32)
    zeros = jnp.zeros((_ZROWS, 128), jnp.float32)

    def body(dst_ref, ones_ref, z_ref, out_ref, dst_v, ones_v, zbuf, acc):
        cid = lax.axis_index("c")
        sid = lax.axis_index("s")
        wid = cid * _NS + sid
        pltpu.sync_copy(dst_ref.at[wid], dst_v)
        pltpu.sync_copy(ones_ref, ones_v)
        pltpu.sync_copy(z_ref, zbuf)
        w0 = sid * _STR
        _zero_stripe(zbuf, acc, w0)
        plsc.subcore_barrier()

        def jbody(j, carry):
            pltpu.sync_copy(ones_v, acc.at[dst_v.at[j]], add=True)
            return carry
        lax.fori_loop(0, _NB, jbody, 0)
        plsc.subcore_barrier()
        pltpu.sync_copy(acc.at[pl.ds(w0, _STR)],
                        out_ref.at[cid, pl.ds(w0, _STR)])

    kfn = pl.kernel(
        body,
        out_type=jax.ShapeDtypeStruct((_NC, _RP, 128), jnp.float32),
        mesh=_mesh,
        scratch_types=[
            pltpu.VMEM((_NB, _BATCH), jnp.int32),
            pltpu.VMEM((_BATCH, 128), jnp.float32),
            pltpu.VMEM((_ZROWS, 128), jnp.float32),
            pltpu.VMEM_SHARED((_RP, 128), jnp.float32),
        ],
    )
    return kfn(dstp, ones, zeros)


_BM = 1000
_GRID = _N // _BM


def _dinv_of(degp_ref):
    deg = 1.0 + degp_ref[0, :, 0] + degp_ref[1, :, 0]
    return lax.rsqrt(deg)


def _kscale_body(x_ref, degp_ref, out_ref):
    dinv = _dinv_of(degp_ref)
    xv = x_ref[...] * dinv[:, None]
    for c in range(2):
        out_ref[c] = xv[:, c * 128:(c + 1) * 128]


def _kscale(x, degp):
    return pl.pallas_call(
        _kscale_body,
        grid=(_GRID,),
        in_specs=[
            pl.BlockSpec((_BM, 256), lambda i: (i, 0)),
            pl.BlockSpec((_NC, _BM, 128), lambda i: (0, i, 0)),
        ],
        out_specs=pl.BlockSpec((2, _BM, 128), lambda i: (0, i, 0)),
        out_shape=jax.ShapeDtypeStruct((2, _N, 128), jnp.float32),
    )(x, degp)


def _k12_body(b_ref, xs_ref, degp_ref, w1_ref, b1_ref, w2_ref, out_ref):
    dinv = _dinv_of(degp_ref)
    h = None
    for c in range(2):
        aggc = (b_ref[0, c] + b_ref[1, c] + xs_ref[c]) * dinv[:, None]
        term = jnp.dot(aggc, w1_ref[c * 128:(c + 1) * 128, :],
                       preferred_element_type=jnp.float32)
        h = term if h is None else h + term
    h = jnp.maximum(h + b1_ref[...], 0.0)
    ppo = jnp.dot(h, w2_ref[...], preferred_element_type=jnp.float32)
    ppo = ppo * dinv[:, None]
    for c in range(4):
        out_ref[c] = ppo[:, c * 128:(c + 1) * 128]


def _k12(B0, xs, degp, W1, b1, W2):
    return pl.pallas_call(
        _k12_body,
        grid=(_GRID,),
        in_specs=[
            pl.BlockSpec((_NC, 2, _BM, 128), lambda i: (0, 0, i, 0)),
            pl.BlockSpec((2, _BM, 128), lambda i: (0, i, 0)),
            pl.BlockSpec((_NC, _BM, 128), lambda i: (0, i, 0)),
            pl.BlockSpec((256, 512), lambda i: (0, 0)),
            pl.BlockSpec((1, 512), lambda i: (0, 0)),
            pl.BlockSpec((512, 512), lambda i: (0, 0)),
        ],
        out_specs=pl.BlockSpec((4, _BM, 128), lambda i: (0, i, 0)),
        out_shape=jax.ShapeDtypeStruct((4, _N, 128), jnp.float32),
    )(B0, xs, degp, W1, b1, W2)


def _mid_body(nc_in, nc_out, b_ref, pp_ref, degp_ref, w_ref, bias_ref, out_ref):
    dinv = _dinv_of(degp_ref)
    acc = None
    for c in range(nc_in):
        hc = b_ref[0, c] + b_ref[1, c] + pp_ref[c]
        hc = jnp.maximum(hc * dinv[:, None] + bias_ref[0, c * 128:(c + 1) * 128], 0.0)
        term = jnp.dot(hc, w_ref[c * 128:(c + 1) * 128, :],
                       preferred_element_type=jnp.float32)
        acc = term if acc is None else acc + term
    ppo = acc * dinv[:, None]
    for c in range(nc_out):
        out_ref[c] = ppo[:, c * 128:(c + 1) * 128]


def _kmid(B, pp, degp, W, bias, nc_in, nc_out):
    return pl.pallas_call(
        functools.partial(_mid_body, nc_in, nc_out),
        grid=(_GRID,),
        in_specs=[
            pl.BlockSpec((_NC, nc_in, _BM, 128), lambda i: (0, 0, i, 0)),
            pl.BlockSpec((nc_in, _BM, 128), lambda i: (0, i, 0)),
            pl.BlockSpec((_NC, _BM, 128), lambda i: (0, i, 0)),
            pl.BlockSpec((nc_in * 128, nc_out * 128), lambda i: (0, 0)),
            pl.BlockSpec((1, nc_in * 128), lambda i: (0, 0)),
        ],
        out_specs=pl.BlockSpec((nc_out, _BM, 128), lambda i: (0, i, 0)),
        out_shape=jax.ShapeDtypeStruct((nc_out, _N, 128), jnp.float32),
    )(B, pp, degp, W, bias)


def _k4_body(b_ref, pp_ref, degp_ref, bias_ref, batch_ref,
             wf1_ref, bf1_ref, wf2_ref, bf2_ref, out_ref, accp, accc):
    i = pl.program_id(0)

    @pl.when(i == 0)
    def _():
        accp[...] = jnp.zeros((_G, 256), jnp.float32)
        accc[...] = jnp.zeros((_G, 128), jnp.float32)

    dinv = _dinv_of(degp_ref)
    gids = lax.broadcasted_iota(jnp.int32, (_G, _BM), 0)
    oh = (batch_ref[0, 0][None, :] == gids).astype(jnp.float32)
    for c in range(2):
        hc = b_ref[0, c] + b_ref[1, c] + pp_ref[c]
        hc = hc * dinv[:, None] + bias_ref[0, c * 128:(c + 1) * 128]
        accp[:, c * 128:(c + 1) * 128] += jnp.dot(
            oh, hc, preferred_element_type=jnp.float32)
    cnt = jnp.sum(oh, axis=1, keepdims=True)
    accc[...] += jnp.broadcast_to(cnt, (_G, 128))

    @pl.when(i == _GRID - 1)
    def _():
        pooled = accp[...] / jnp.maximum(accc[:, 0:1], 1.0)
        f = jnp.dot(pooled, wf1_ref[...], preferred_element_type=jnp.float32)
        f = jnp.maximum(f + bf1_ref[...], 0.0)
        f2 = jnp.dot(f, wf2_ref[...], preferred_element_type=jnp.float32)
        f2 = f2 + bf2_ref[...]
        m = jnp.max(f2, axis=1, keepdims=True)
        lse = jnp.log(jnp.sum(jnp.exp(f2 - m), axis=1, keepdims=True)) + m
        out_ref[...] = f2 - lse


def _k4(B3, pp3, degp, bias3, batch_r, Wf1, bf1, Wf2, bf2):
    return pl.pallas_call(
        _k4_body,
        grid=(_GRID,),
        in_specs=[
            pl.BlockSpec((_NC, 2, _BM, 128), lambda i: (0, 0, i, 0)),
            pl.BlockSpec((2, _BM, 128), lambda i: (0, i, 0)),
            pl.BlockSpec((_NC, _BM, 128), lambda i: (0, i, 0)),
            pl.BlockSpec((1, 256), lambda i: (0, 0)),
            pl.BlockSpec((1, 1, _BM), lambda i: (i, 0, 0)),
            pl.BlockSpec((256, 512), lambda i: (0, 0)),
            pl.BlockSpec((1, 512), lambda i: (0, 0)),
            pl.BlockSpec((512, 128), lambda i: (0, 0)),
            pl.BlockSpec((1, 128), lambda i: (0, 0)),
        ],
        out_specs=pl.BlockSpec((_G, 128), lambda i: (0, 0)),
        out_shape=jax.ShapeDtypeStruct((_G, 128), jnp.float32),
        scratch_shapes=[
            pltpu.VMEM((_G, 256), jnp.float32),
            pltpu.VMEM((_G, 128), jnp.float32),
        ],
    )(B3, pp3, degp, bias3, batch_r, Wf1, bf1, Wf2, bf2)


def kernel(x, edge_index, batch, W1, b1, W2, b2, W3, b3, Wf1, bf1, Wf2, bf2):
    src, dst = edge_index[0], edge_index[1]
    npad = _EPAD - _E
    srcp = jnp.concatenate([src, jnp.zeros((npad,), jnp.int32)]
                           ).reshape(_NW, _NB, _BATCH)
    dstp = jnp.concatenate([dst, jnp.full((npad,), _N, jnp.int32)]
                           ).reshape(_NW, _NB, _BATCH)

    degp = _deg_sc(dstp)

    xs = _kscale(x, degp)
    B0 = _seg_sum_sc(xs, srcp, dstp, 2)
    pp2 = _k12(B0, xs, degp, W1, b1.reshape(1, 512), W2)
    B2 = _seg_sum_sc(pp2, srcp, dstp, 4)
    pp3 = _kmid(B2, pp2, degp, W3, b2.reshape(1, 512), 4, 2)
    B3 = _seg_sum_sc(pp3, srcp, dstp, 2)

    batch_r = batch.reshape(_GRID, 1, _BM)
    return _k4(B3, pp3, degp, b3.reshape(1, 256), batch_r,
               Wf1, bf1.reshape(1, 512), Wf2, bf2.reshape(1, 128))
